# baseline (speedup 1.0000x reference)
import functools

import jax
import jax.numpy as jnp
from jax import lax
from jax.experimental import pallas as pl
from jax.experimental.pallas import tpu as pltpu
from jax.experimental.pallas import tpu_sc as plsc

_NC = 2
_NS = 16
_NW = _NC * _NS

_LANES = 128
_CHUNK = 128
_PEROWS = 32
_WAVE = 2


def _positional_encoding(seqlen: int, dmodel: int) -> jnp.ndarray:
    pos = jnp.arange(seqlen, dtype=jnp.float32)[:, None]
    ch = jnp.arange(dmodel, dtype=jnp.float32)[None, :]
    angle = pos * jnp.power(10000.0, -2.0 * ch / float(dmodel))
    even_mask = (jnp.arange(dmodel) % 2 == 0)[None, :]
    return jnp.where(even_mask, jnp.sin(angle), jnp.cos(angle))


@functools.partial(jax.jit, static_argnames=("n_rows", "batch"))
def _sc_embed(idx_flat, table_pad, pe_bcast, *, n_rows, batch):
    b_per_w = n_rows // _NW
    n_chunks = b_per_w // _CHUNK
    seqlen = pe_bcast.shape[0]
    mesh = plsc.VectorSubcoreMesh(core_axis_name="c", subcore_axis_name="s")

    @functools.partial(
        pl.kernel,
        out_type=jax.ShapeDtypeStruct((n_rows, _LANES), jnp.float32),
        mesh=mesh,
        scratch_types=[
            pltpu.VMEM((b_per_w,), jnp.int32),
            pltpu.VMEM_SHARED((seqlen // _NC, _PEROWS, _LANES), jnp.float32),
            [pltpu.VMEM((_CHUNK, _LANES), jnp.float32)
             for _ in range(2 * _WAVE)],
            [pltpu.SemaphoreType.DMA for _ in range(2 * _WAVE)],
            [pltpu.SemaphoreType.DMA for _ in range(2 * _WAVE)],
        ],
    )
    def body(table_hbm, idx_hbm, pe_hbm, out_hbm,
             idx_v, pe_sh, rows_v, gsems, osems):
        sid = lax.axis_index("s")
        cid = lax.axis_index("c")
        wid = cid * _NS + sid
        base = wid * b_per_w
        l_half = seqlen // _NC

        @pl.when(sid == 0)
        def _():
            pltpu.sync_copy(pe_hbm.at[pl.ds(cid * l_half, l_half)], pe_sh)

        plsc.subcore_barrier()
        pltpu.sync_copy(idx_hbm.at[pl.ds(base, b_per_w)], idx_v)

        def fire(b, c):
            off = base + c * _CHUNK
            l_loc = off // batch - cid * l_half
            for r in range(_CHUNK // _PEROWS):
                pltpu.sync_copy(pe_sh.at[l_loc],
                                rows_v[b].at[pl.ds(r * _PEROWS, _PEROWS)])
            pltpu.async_copy(
                table_hbm.at[idx_v.at[pl.ds(c * _CHUNK, _CHUNK)]],
                rows_v[b], gsems[b], add=True)

        def drain_gather(b):
            pltpu.make_async_copy(out_hbm.at[pl.ds(0, _CHUNK)],
                                  rows_v[b], gsems[b]).wait()

        def store(b, c):
            return pltpu.async_copy(
                rows_v[b], out_hbm.at[pl.ds(base + c * _CHUNK, _CHUNK)],
                osems[b])

        for b in range(_WAVE):
            fire(b, b)

        def body(g):
            for i in range(_WAVE):
                fire(_WAVE + i, g + _WAVE + i)
            for b in range(_WAVE):
                drain_gather(b)
            stores_a = [store(b, g + b) for b in range(_WAVE)]
            for s in stores_a:
                s.wait()
            for b in range(_WAVE):
                nxt = g + 2 * _WAVE + b

                @pl.when(nxt < n_chunks)
                def _():
                    fire(b, nxt)

            for i in range(_WAVE):
                drain_gather(_WAVE + i)
            stores_b = [store(_WAVE + i, g + _WAVE + i) for i in range(_WAVE)]
            for s in stores_b:
                s.wait()

        pl.loop(0, n_chunks, step=2 * _WAVE)(body)

    return body(table_pad, idx_flat, pe_bcast)


def kernel(seq, table):
    batch, seqlen = seq.shape
    dmodel = table.shape[1]
    n_rows = batch * seqlen
    idx_flat = jnp.transpose(seq).reshape(n_rows).astype(jnp.int32)
    table_pad = jnp.pad(table, ((0, 0), (0, _LANES - dmodel)))
    pe = _positional_encoding(seqlen, dmodel)
    pe_pad = jnp.pad(pe, ((0, 0), (0, _LANES - dmodel)))
    pe_bcast = jnp.broadcast_to(pe_pad[:, None, :], (seqlen, _PEROWS, _LANES))
    out = _sc_embed(idx_flat, table_pad, pe_bcast, n_rows=n_rows, batch=batch)
    out = out[:, :dmodel].reshape(seqlen, batch, dmodel)
    return jnp.transpose(out, (1, 0, 2))


# speedup vs baseline: 1.0222x; 1.0222x over previous
import functools

import jax
import jax.numpy as jnp
from jax import lax
from jax.experimental import pallas as pl
from jax.experimental.pallas import tpu as pltpu
from jax.experimental.pallas import tpu_sc as plsc

_NC = 2
_NS = 16
_NW = _NC * _NS

_LANES = 128
_CHUNK = 64
_PEROWS = 32
_WAVE = 4


def _positional_encoding(seqlen: int, dmodel: int) -> jnp.ndarray:
    pos = jnp.arange(seqlen, dtype=jnp.float32)[:, None]
    ch = jnp.arange(dmodel, dtype=jnp.float32)[None, :]
    angle = pos * jnp.power(10000.0, -2.0 * ch / float(dmodel))
    even_mask = (jnp.arange(dmodel) % 2 == 0)[None, :]
    return jnp.where(even_mask, jnp.sin(angle), jnp.cos(angle))


@functools.partial(jax.jit, static_argnames=("n_rows", "batch"))
def _sc_embed(idx_flat, table_pad, pe_bcast, *, n_rows, batch):
    b_per_w = n_rows // _NW
    n_chunks = b_per_w // _CHUNK
    seqlen = pe_bcast.shape[0]
    mesh = plsc.VectorSubcoreMesh(core_axis_name="c", subcore_axis_name="s")

    @functools.partial(
        pl.kernel,
        out_type=jax.ShapeDtypeStruct((n_rows, _LANES), jnp.float32),
        mesh=mesh,
        scratch_types=[
            pltpu.VMEM((b_per_w,), jnp.int32),
            pltpu.VMEM_SHARED((seqlen // _NC, _PEROWS, _LANES), jnp.float32),
            [pltpu.VMEM((_CHUNK, _LANES), jnp.float32)
             for _ in range(2 * _WAVE)],
            [pltpu.SemaphoreType.DMA for _ in range(2 * _WAVE)],
            [pltpu.SemaphoreType.DMA for _ in range(2 * _WAVE)],
        ],
    )
    def body(table_hbm, idx_hbm, pe_hbm, out_hbm,
             idx_v, pe_sh, rows_v, gsems, osems):
        sid = lax.axis_index("s")
        cid = lax.axis_index("c")
        wid = cid * _NS + sid
        base = wid * b_per_w
        l_half = seqlen // _NC

        @pl.when(sid == 0)
        def _():
            pltpu.sync_copy(pe_hbm.at[pl.ds(cid * l_half, l_half)], pe_sh)

        plsc.subcore_barrier()
        pltpu.sync_copy(idx_hbm.at[pl.ds(base, b_per_w)], idx_v)

        def fire(b, c):
            off = base + c * _CHUNK
            l_loc = off // batch - cid * l_half
            fills = [
                pltpu.async_copy(pe_sh.at[l_loc],
                                 rows_v[b].at[pl.ds(r * _PEROWS, _PEROWS)],
                                 osems[b])
                for r in range(_CHUNK // _PEROWS)]
            for f in fills:
                f.wait()
            pltpu.async_copy(
                table_hbm.at[idx_v.at[pl.ds(c * _CHUNK, _CHUNK)]],
                rows_v[b], gsems[b], add=True)

        def drain_gather(b):
            pltpu.make_async_copy(out_hbm.at[pl.ds(0, _CHUNK)],
                                  rows_v[b], gsems[b]).wait()

        def store(b, c):
            return pltpu.async_copy(
                rows_v[b], out_hbm.at[pl.ds(base + c * _CHUNK, _CHUNK)],
                osems[b])

        for b in range(_WAVE):
            fire(b, b)

        def body(g):
            for i in range(_WAVE):
                fire(_WAVE + i, g + _WAVE + i)
            for b in range(_WAVE):
                drain_gather(b)
            stores_a = [store(b, g + b) for b in range(_WAVE)]
            for s in stores_a:
                s.wait()
            for b in range(_WAVE):
                nxt = g + 2 * _WAVE + b

                @pl.when(nxt < n_chunks)
                def _():
                    fire(b, nxt)

            for i in range(_WAVE):
                drain_gather(_WAVE + i)
            stores_b = [store(_WAVE + i, g + _WAVE + i) for i in range(_WAVE)]
            for s in stores_b:
                s.wait()

        pl.loop(0, n_chunks, step=2 * _WAVE)(body)

    return body(table_pad, idx_flat, pe_bcast)


def kernel(seq, table):
    batch, seqlen = seq.shape
    dmodel = table.shape[1]
    n_rows = batch * seqlen
    idx_flat = jnp.transpose(seq).reshape(n_rows).astype(jnp.int32)
    table_pad = jnp.pad(table, ((0, 0), (0, _LANES - dmodel)))
    pe = _positional_encoding(seqlen, dmodel)
    pe_pad = jnp.pad(pe, ((0, 0), (0, _LANES - dmodel)))
    pe_bcast = jnp.broadcast_to(pe_pad[:, None, :], (seqlen, _PEROWS, _LANES))
    out = _sc_embed(idx_flat, table_pad, pe_bcast, n_rows=n_rows, batch=batch)
    out = out[:, :dmodel].reshape(seqlen, batch, dmodel)
    return jnp.transpose(out, (1, 0, 2))
